# baseline (speedup 1.0000x reference)
import jax
import jax.numpy as jnp
from jax import lax
from jax.experimental import pallas as pl
from jax.experimental.pallas import tpu as pltpu
from jax.experimental.pallas import tpu_sc as plsc

VOCAB = 32000
DIM = 4096
XROWS = 4
XCOLS = 2048
B = XROWS * XCOLS

NUM_CORES = 2
NUM_SUBCORES = 16
NW = NUM_CORES * NUM_SUBCORES
B_PER_W = B // NW
W_PER_XROW = XCOLS // B_PER_W
CHUNK = 8
NBUF = 3
NCHUNK = B_PER_W // CHUNK


def _emb_body(table_hbm, x_hbm, out_hbm, idx_v, rows, gsems, ssems):
    wid = lax.axis_index("s") * NUM_CORES + lax.axis_index("c")
    xr = wid // W_PER_XROW
    c0 = (wid % W_PER_XROW) * B_PER_W

    pltpu.sync_copy(x_hbm.at[xr, pl.ds(c0, B_PER_W)], idx_v)

    def gather(g, s):
        pltpu.async_copy(
            table_hbm.at[idx_v.at[pl.ds(g * CHUNK, CHUNK)]],
            rows[s], gsems[s])

    def put(g, s):
        pltpu.async_copy(
            rows[s], out_hbm.at[xr, pl.ds(c0 + g * CHUNK, CHUNK)], ssems[s])

    def wait_gather(s):
        pltpu.make_async_copy(
            table_hbm.at[idx_v.at[pl.ds(0, CHUNK)]], rows[s], gsems[s]).wait()

    def wait_put(s):
        pltpu.make_async_copy(
            rows[s], out_hbm.at[xr, pl.ds(c0, CHUNK)], ssems[s]).wait()

    for g in range(NBUF):
        gather(g, g % NBUF)

    for g in range(NCHUNK):
        s = g % NBUF
        wait_gather(s)
        put(g, s)
        gn = g + 2
        if NBUF <= gn < NCHUNK:
            sn = gn % NBUF
            wait_put(sn)
            gather(gn, sn)

    for g in range(NCHUNK - NBUF, NCHUNK):
        wait_put(g % NBUF)


@jax.jit
def _embedding_lookup(weight, x):
    mesh = plsc.VectorSubcoreMesh(
        core_axis_name="c", subcore_axis_name="s",
        num_cores=NUM_CORES, num_subcores=NUM_SUBCORES,
    )
    return pl.kernel(
        _emb_body,
        out_type=jax.ShapeDtypeStruct((XROWS, XCOLS, DIM), jnp.float32),
        mesh=mesh,
        scratch_types=[
            pltpu.VMEM((B_PER_W,), jnp.int32),
            [pltpu.VMEM((CHUNK, DIM), jnp.float32) for _ in range(NBUF)],
            [pltpu.SemaphoreType.DMA for _ in range(NBUF)],
            [pltpu.SemaphoreType.DMA for _ in range(NBUF)],
        ],
    )(weight, x)


def kernel(x, weight):
    return _embedding_lookup(weight, x)


# speedup vs baseline: 1.0050x; 1.0050x over previous
import jax
import jax.numpy as jnp
from jax import lax
from jax.experimental import pallas as pl
from jax.experimental.pallas import tpu as pltpu
from jax.experimental.pallas import tpu_sc as plsc

VOCAB = 32000
DIM = 4096
XROWS = 4
XCOLS = 2048
B = XROWS * XCOLS

NUM_CORES = 2
NUM_SUBCORES = 16
NW = NUM_CORES * NUM_SUBCORES
B_PER_W = B // NW
W_PER_XROW = XCOLS // B_PER_W
CHUNK = 8
NBUF = 3
NCHUNK = B_PER_W // CHUNK


def _emb_body(table_hbm, x_hbm, out_hbm, idx_v, rows, gsems, ssems):
    wid = lax.axis_index("s") * NUM_CORES + lax.axis_index("c")
    xr = wid // W_PER_XROW
    c0 = (wid % W_PER_XROW) * B_PER_W

    pltpu.sync_copy(x_hbm.at[xr, pl.ds(c0, B_PER_W)], idx_v)

    def gather(g, s):
        pltpu.async_copy(
            table_hbm.at[idx_v.at[pl.ds(g * CHUNK, CHUNK)]],
            rows[s], gsems[s])

    def put(g, s):
        pltpu.async_copy(
            rows[s], out_hbm.at[xr, pl.ds(c0 + g * CHUNK, CHUNK)], ssems[s])

    def wait_gather(s):
        pltpu.make_async_copy(
            table_hbm.at[idx_v.at[pl.ds(0, CHUNK)]], rows[s], gsems[s]).wait()

    def wait_put(s):
        pltpu.make_async_copy(
            rows[s], out_hbm.at[xr, pl.ds(c0, CHUNK)], ssems[s]).wait()

    for g in range(NBUF):
        gather(g, g % NBUF)

    for g in range(NCHUNK):
        s = g % NBUF
        wait_gather(s)
        put(g, s)
        gn = g + 2
        if NBUF <= gn < NCHUNK:
            sn = gn % NBUF
            wait_put(sn)
            gather(gn, sn)

    for g in range(NCHUNK - NBUF, NCHUNK):
        wait_put(g % NBUF)


@jax.jit
def _embedding_lookup(weight, x):
    mesh = plsc.VectorSubcoreMesh(
        core_axis_name="c", subcore_axis_name="s",
        num_cores=NUM_CORES, num_subcores=NUM_SUBCORES,
    )
    return pl.kernel(
        _emb_body,
        out_type=jax.ShapeDtypeStruct((XROWS, XCOLS, DIM), jnp.float32),
        mesh=mesh,
        scratch_types=[
            pltpu.VMEM((B_PER_W,), jnp.int32),
            [pltpu.VMEM((CHUNK, DIM), jnp.float32) for _ in range(NBUF)],
            [pltpu.SemaphoreType.DMA for _ in range(NBUF)],
            [pltpu.SemaphoreType.DMA for _ in range(NBUF)],
        ],
    )(weight, x)


def kernel(x, weight):
    return _embedding_lookup(weight, x.astype(jnp.int32))
